# Initial kernel scaffold; baseline (speedup 1.0000x reference)
import jax
import jax.numpy as jnp
from jax.experimental import pallas as pl


def kernel(input_ids, embed_table, W, b):
    raise NotImplementedError("write your pallas kernel here")


# speedup vs baseline: 1.7927x; 1.7927x over previous
import functools

import jax
import jax.numpy as jnp
from jax import lax
from jax.experimental import pallas as pl
from jax.experimental.pallas import tpu as pltpu
from jax.experimental.pallas import tpu_sc as plsc

VOCAB = 32
N_IDS = 4096 * 200
NW = 32
PER_W = N_IDS // NW
CHUNK = 1024
N_CH = PER_W // CHUNK


def _table_body(e_ref, wt_ref, b_ref, t_ref):
    t_ref[...] = (
        jnp.dot(e_ref[...], wt_ref[...], preferred_element_type=jnp.float32)
        + b_ref[...]
    )


def _make_table(embed_table, wt, b2):
    return pl.pallas_call(
        _table_body,
        out_shape=jax.ShapeDtypeStruct((VOCAB, VOCAB), jnp.float32),
    )(embed_table, wt, b2)


def _gather_body(t_hbm, ids_hbm, out_hbm, idx_v, rows_v, sem):
    wid = lax.axis_index("s") * 2 + lax.axis_index("c")
    base = wid * PER_W

    def step(i, carry):
        off = base + i * CHUNK
        pltpu.sync_copy(ids_hbm.at[pl.ds(off, CHUNK)], idx_v)
        pltpu.async_copy(t_hbm.at[idx_v], rows_v, sem).wait()
        pltpu.sync_copy(rows_v, out_hbm.at[pl.ds(off, CHUNK)])
        return carry

    lax.fori_loop(0, N_CH, step, 0)


_gather = functools.partial(
    pl.kernel,
    out_type=jax.ShapeDtypeStruct((N_IDS, VOCAB), jnp.float32),
    mesh=plsc.VectorSubcoreMesh(core_axis_name="c", subcore_axis_name="s"),
    scratch_types=[
        pltpu.VMEM((CHUNK,), jnp.int32),
        pltpu.VMEM((CHUNK, VOCAB), jnp.float32),
        pltpu.SemaphoreType.DMA,
    ],
    compiler_params=pltpu.CompilerParams(use_tc_tiling_on_sc=False),
)(_gather_body)


def kernel(input_ids, embed_table, W, b):
    table = _make_table(embed_table, W.T, b.reshape(1, VOCAB))
    ids = input_ids.reshape(N_IDS).astype(jnp.int32)
    out = _gather(table, ids)
    return out.reshape(input_ids.shape[0], input_ids.shape[1], VOCAB)


# speedup vs baseline: 1.8072x; 1.0081x over previous
import functools

import jax
import jax.numpy as jnp
from jax import lax
from jax.experimental import pallas as pl
from jax.experimental.pallas import tpu as pltpu
from jax.experimental.pallas import tpu_sc as plsc

VOCAB = 32
N_IDS = 4096 * 200
NW = 32
PER_W = N_IDS // NW
CHUNK = 1600
N_CH = PER_W // CHUNK


def _table_body(e_ref, wt_ref, b_ref, t_ref):
    t_ref[...] = (
        jnp.dot(e_ref[...], wt_ref[...], preferred_element_type=jnp.float32)
        + b_ref[...]
    )


def _make_table(embed_table, wt, b2):
    return pl.pallas_call(
        _table_body,
        out_shape=jax.ShapeDtypeStruct((VOCAB, VOCAB), jnp.float32),
    )(embed_table, wt, b2)


def _gather_body(t_hbm, ids_hbm, out_hbm, idx_v, rows0, rows1, g0, g1, s0, s1):
    wid = lax.axis_index("s") * 2 + lax.axis_index("c")
    base = wid * PER_W

    pltpu.sync_copy(ids_hbm.at[pl.ds(base, PER_W)], idx_v)

    def gather(i, buf, sem):
        return pltpu.make_async_copy(
            t_hbm.at[idx_v.at[pl.ds(i * CHUNK, CHUNK)]], buf, sem
        )

    def store(i, buf, sem):
        return pltpu.make_async_copy(
            buf, out_hbm.at[pl.ds(base + i * CHUNK, CHUNK)], sem
        )

    gather(0, rows0, g0).start()

    def pair(p, carry):
        i0 = 2 * p
        i1 = 2 * p + 1

        @pl.when(p > 0)
        def _():
            store(i0 - 1, rows1, s1).wait()

        gather(i1, rows1, g1).start()
        gather(i0, rows0, g0).wait()
        store(i0, rows0, s0).start()

        @pl.when(p + 1 < N_CH // 2)
        def _():
            store(i0, rows0, s0).wait()
            gather(i1 + 1, rows0, g0).start()

        gather(i1, rows1, g1).wait()
        store(i1, rows1, s1).start()
        return carry

    lax.fori_loop(0, N_CH // 2, pair, 0)

    store(N_CH - 2, rows0, s0).wait()
    store(N_CH - 1, rows1, s1).wait()


_gather = functools.partial(
    pl.kernel,
    out_type=jax.ShapeDtypeStruct((N_IDS, VOCAB), jnp.float32),
    mesh=plsc.VectorSubcoreMesh(core_axis_name="c", subcore_axis_name="s"),
    scratch_types=[
        pltpu.VMEM((PER_W,), jnp.int32),
        pltpu.VMEM((CHUNK, VOCAB), jnp.float32),
        pltpu.VMEM((CHUNK, VOCAB), jnp.float32),
        pltpu.SemaphoreType.DMA,
        pltpu.SemaphoreType.DMA,
        pltpu.SemaphoreType.DMA,
        pltpu.SemaphoreType.DMA,
    ],
    compiler_params=pltpu.CompilerParams(use_tc_tiling_on_sc=False),
)(_gather_body)


def kernel(input_ids, embed_table, W, b):
    table = _make_table(embed_table, W.T, b.reshape(1, VOCAB))
    ids = input_ids.reshape(N_IDS).astype(jnp.int32)
    out = _gather(table, ids)
    return out.reshape(input_ids.shape[0], input_ids.shape[1], VOCAB)


# speedup vs baseline: 4.5812x; 2.5350x over previous
import functools

import jax
import jax.numpy as jnp
from jax import lax
from jax.experimental import pallas as pl
from jax.experimental.pallas import tpu as pltpu
from jax.experimental.pallas import tpu_sc as plsc

VOCAB = 32
N_IDS = 4096 * 200
NW = 32
PER_W = N_IDS // NW
CHUNK = 1600
N_CH = PER_W // CHUNK


def _table_body(e_ref, wt_ref, b_ref, t_ref):
    t = (
        jnp.dot(e_ref[...], wt_ref[...], preferred_element_type=jnp.float32)
        + b_ref[...]
    )
    t_ref[...] = jnp.broadcast_to(t[None], (NW, VOCAB, VOCAB))


def _make_table(embed_table, wt, b2):
    return pl.pallas_call(
        _table_body,
        out_shape=jax.ShapeDtypeStruct((NW, VOCAB, VOCAB), jnp.float32),
    )(embed_table, wt, b2)


def _gather_body(t_hbm, ids_hbm, out_hbm, idx_v, rows0, rows1, g0, g1, s0, s1):
    wid = lax.axis_index("s") * 2 + lax.axis_index("c")
    base = wid * PER_W
    t_mine = t_hbm.at[wid]

    pltpu.sync_copy(ids_hbm.at[pl.ds(base, PER_W)], idx_v)

    def gather(i, buf, sem):
        return pltpu.make_async_copy(
            t_mine.at[idx_v.at[pl.ds(i * CHUNK, CHUNK)]], buf, sem
        )

    def store(i, buf, sem):
        return pltpu.make_async_copy(
            buf, out_hbm.at[pl.ds(base + i * CHUNK, CHUNK)], sem
        )

    gather(0, rows0, g0).start()

    def pair(p, carry):
        i0 = 2 * p
        i1 = 2 * p + 1

        @pl.when(p > 0)
        def _():
            store(i0 - 1, rows1, s1).wait()

        gather(i1, rows1, g1).start()
        gather(i0, rows0, g0).wait()
        store(i0, rows0, s0).start()

        @pl.when(p + 1 < N_CH // 2)
        def _():
            store(i0, rows0, s0).wait()
            gather(i1 + 1, rows0, g0).start()

        gather(i1, rows1, g1).wait()
        store(i1, rows1, s1).start()
        return carry

    lax.fori_loop(0, N_CH // 2, pair, 0)

    store(N_CH - 2, rows0, s0).wait()
    store(N_CH - 1, rows1, s1).wait()


_gather = functools.partial(
    pl.kernel,
    out_type=jax.ShapeDtypeStruct((N_IDS, VOCAB), jnp.float32),
    mesh=plsc.VectorSubcoreMesh(core_axis_name="c", subcore_axis_name="s"),
    scratch_types=[
        pltpu.VMEM((PER_W,), jnp.int32),
        pltpu.VMEM((CHUNK, VOCAB), jnp.float32),
        pltpu.VMEM((CHUNK, VOCAB), jnp.float32),
        pltpu.SemaphoreType.DMA,
        pltpu.SemaphoreType.DMA,
        pltpu.SemaphoreType.DMA,
        pltpu.SemaphoreType.DMA,
    ],
    compiler_params=pltpu.CompilerParams(use_tc_tiling_on_sc=False),
)(_gather_body)


def kernel(input_ids, embed_table, W, b):
    table = _make_table(embed_table, W.T, b.reshape(1, VOCAB))
    ids = input_ids.reshape(N_IDS).astype(jnp.int32)
    out = _gather(table, ids)
    return out.reshape(input_ids.shape[0], input_ids.shape[1], VOCAB)
